# baseline (speedup 1.0000x reference)
import functools

import jax
import jax.numpy as jnp
from jax import lax
from jax.experimental import pallas as pl
from jax.experimental.pallas import tpu as pltpu
from jax.experimental.pallas import tpu_sc as plsc

BATCH = 16384
F = 512
LANES = 16
NC = 2
NS = 16
NW = NC * NS
ROWS_PER_W = BATCH // NW
R = 64
NCHUNK = ROWS_PER_W // R
NJ = F // LANES

_mesh = plsc.VectorSubcoreMesh(core_axis_name="c", subcore_axis_name="s")


@functools.partial(
    pl.kernel,
    out_type=jax.ShapeDtypeStruct((BATCH * F,), jnp.float32),
    mesh=_mesh,
    compiler_params=pltpu.CompilerParams(needs_layout_passes=False),
    scratch_types=[
        pltpu.VMEM((F,), jnp.int32),
        pltpu.VMEM((R * F,), jnp.float32),
        pltpu.VMEM((R * F,), jnp.float32),
    ],
)
def _sc_gather(x_hbm, idx_hbm, out_hbm, idx_v, xbuf, obuf):
    wid = lax.axis_index("s") * NC + lax.axis_index("c")
    base = wid * ROWS_PER_W * F

    pltpu.sync_copy(idx_hbm, idx_v)
    cols = [idx_v[pl.ds(j * LANES, LANES)] for j in range(NJ)]

    def chunk_body(ci, carry):
        off = base + ci * (R * F)
        pltpu.sync_copy(x_hbm.at[pl.ds(off, R * F)], xbuf)

        def row_body(r, c):
            rbase = jnp.full((LANES,), r * F, dtype=jnp.int32)
            for j in range(NJ):
                vals = plsc.load_gather(xbuf, [cols[j] + rbase])
                obuf[pl.ds(r * F + j * LANES, LANES)] = vals
            return c

        lax.fori_loop(0, R, row_body, 0)
        pltpu.sync_copy(obuf, out_hbm.at[pl.ds(off, R * F)])
        return carry

    lax.fori_loop(0, NCHUNK, chunk_body, 0)


def kernel(x, sparse_index):
    out_flat = _sc_gather(x.reshape(-1), sparse_index.astype(jnp.int32))
    return out_flat.reshape(BATCH, F)


# speedup vs baseline: 1.6051x; 1.6051x over previous
import functools

import jax
import jax.numpy as jnp
from jax import lax
from jax.experimental import pallas as pl
from jax.experimental.pallas import tpu as pltpu
from jax.experimental.pallas import tpu_sc as plsc

BATCH = 16384
F = 512
LANES = 16
NC = 2
NS = 16
NW = NC * NS
ROWS_PER_W = BATCH // NW
R = 32
CHUNK = R * F
NCHUNK = ROWS_PER_W // R
NPAIR = NCHUNK // 2
NJ = F // LANES

_mesh = plsc.VectorSubcoreMesh(core_axis_name="c", subcore_axis_name="s")


@functools.partial(
    pl.kernel,
    out_type=jax.ShapeDtypeStruct((BATCH * F,), jnp.float32),
    mesh=_mesh,
    compiler_params=pltpu.CompilerParams(needs_layout_passes=False),
    scratch_types=[
        pltpu.VMEM((F,), jnp.int32),
        pltpu.VMEM((CHUNK,), jnp.float32),
        pltpu.VMEM((CHUNK,), jnp.float32),
        pltpu.VMEM((CHUNK,), jnp.float32),
        pltpu.VMEM((CHUNK,), jnp.float32),
        pltpu.SemaphoreType.DMA,
        pltpu.SemaphoreType.DMA,
        pltpu.SemaphoreType.DMA,
        pltpu.SemaphoreType.DMA,
    ],
)
def _sc_gather(x_hbm, idx_hbm, out_hbm, idx_v, xb0, xb1, ob0, ob1,
               si0, si1, so0, so1):
    wid = lax.axis_index("s") * NC + lax.axis_index("c")
    base = wid * ROWS_PER_W * F

    pltpu.sync_copy(idx_hbm, idx_v)
    cols = [idx_v[pl.ds(j * LANES, LANES)] for j in range(NJ)]

    def start_in(ci, buf, sem):
        pltpu.async_copy(x_hbm.at[pl.ds(base + ci * CHUNK, CHUNK)], buf, sem)

    def start_out(ci, buf, sem):
        pltpu.async_copy(buf, out_hbm.at[pl.ds(base + ci * CHUNK, CHUNK)], sem)

    def wait_in(buf, sem):
        pltpu.make_async_copy(x_hbm.at[pl.ds(base, CHUNK)], buf, sem).wait()

    def wait_out(buf, sem):
        pltpu.make_async_copy(buf, out_hbm.at[pl.ds(base, CHUNK)], sem).wait()

    def gather(xb, ob):
        @plsc.parallel_loop(0, R, unroll=4)
        def _row(r):
            rbase = jnp.full((LANES,), r * F, dtype=jnp.int32)
            for j in range(NJ):
                vals = plsc.load_gather(xb, [cols[j] + rbase])
                ob[pl.ds(r * F + j * LANES, LANES)] = vals

    start_in(0, xb0, si0)
    start_in(1, xb1, si1)

    def pair_body(g, carry):
        for b, (xb, ob, si, so) in enumerate(
            ((xb0, ob0, si0, so0), (xb1, ob1, si1, so1))):
            ci = 2 * g + b
            wait_in(xb, si)

            @pl.when(g > 0)
            def _():
                wait_out(ob, so)

            gather(xb, ob)
            start_out(ci, ob, so)

            @pl.when(g < NPAIR - 1)
            def _():
                start_in(ci + 2, xb, si)
        return carry

    lax.fori_loop(0, NPAIR, pair_body, 0)

    wait_out(ob0, so0)
    wait_out(ob1, so1)


def kernel(x, sparse_index):
    out_flat = _sc_gather(x.reshape(-1), sparse_index.astype(jnp.int32))
    return out_flat.reshape(BATCH, F)


# speedup vs baseline: 3.1320x; 1.9512x over previous
import functools

import jax
import jax.numpy as jnp
from jax import lax
from jax.experimental import pallas as pl
from jax.experimental.pallas import tpu as pltpu
from jax.experimental.pallas import tpu_sc as plsc

BATCH = 16384
F = 512
LANES = 16
NC = 2
NS = 16
NW = NC * NS
ROWS_PER_W = BATCH // NW
R = 32
NCHUNK = ROWS_PER_W // R
NPAIR = NCHUNK // 2
NJ = F // LANES

_mesh = plsc.VectorSubcoreMesh(core_axis_name="c", subcore_axis_name="s")


@functools.partial(
    pl.kernel,
    out_type=jax.ShapeDtypeStruct((BATCH, F), jnp.float32),
    mesh=_mesh,
    compiler_params=pltpu.CompilerParams(needs_layout_passes=False),
    scratch_types=[
        pltpu.VMEM((F,), jnp.int32),
        pltpu.VMEM((R, F), jnp.float32),
        pltpu.VMEM((R, F), jnp.float32),
        pltpu.VMEM((R, F), jnp.float32),
        pltpu.VMEM((R, F), jnp.float32),
        pltpu.SemaphoreType.DMA,
        pltpu.SemaphoreType.DMA,
        pltpu.SemaphoreType.DMA,
        pltpu.SemaphoreType.DMA,
    ],
)
def _sc_gather(x_hbm, idx_hbm, out_hbm, idx_v, xb0, xb1, ob0, ob1,
               si0, si1, so0, so1):
    wid = lax.axis_index("s") * NC + lax.axis_index("c")
    base = wid * ROWS_PER_W

    pltpu.sync_copy(idx_hbm, idx_v)
    cols = [idx_v[pl.ds(j * LANES, LANES)] for j in range(NJ)]

    def start_in(ci, buf, sem):
        pltpu.async_copy(x_hbm.at[pl.ds(base + ci * R, R)], buf, sem)

    def start_out(ci, buf, sem):
        pltpu.async_copy(buf, out_hbm.at[pl.ds(base + ci * R, R)], sem)

    def wait_in(buf, sem):
        pltpu.make_async_copy(x_hbm.at[pl.ds(base, R)], buf, sem).wait()

    def wait_out(buf, sem):
        pltpu.make_async_copy(buf, out_hbm.at[pl.ds(base, R)], sem).wait()

    def gather(xb, ob):
        @plsc.parallel_loop(0, R, unroll=4)
        def _row(r):
            row = jnp.full((LANES,), r, dtype=jnp.int32)
            for j in range(NJ):
                vals = plsc.load_gather(xb, [row, cols[j]])
                ob[r, pl.ds(j * LANES, LANES)] = vals

    start_in(0, xb0, si0)
    start_in(1, xb1, si1)

    def pair_body(g, carry):
        for b, (xb, ob, si, so) in enumerate(
            ((xb0, ob0, si0, so0), (xb1, ob1, si1, so1))):
            ci = 2 * g + b
            wait_in(xb, si)

            @pl.when(g > 0)
            def _():
                wait_out(ob, so)

            gather(xb, ob)
            start_out(ci, ob, so)

            @pl.when(g < NPAIR - 1)
            def _():
                start_in(ci + 2, xb, si)
        return carry

    lax.fori_loop(0, NPAIR, pair_body, 0)

    wait_out(ob0, so0)
    wait_out(ob1, so1)


def kernel(x, sparse_index):
    return _sc_gather(x, sparse_index.astype(jnp.int32))
